# baseline (speedup 1.0000x reference)
import functools

import jax
import jax.numpy as jnp
from jax import lax
from jax.experimental import pallas as pl
from jax.experimental.pallas import tpu as pltpu
from jax.experimental.pallas import tpu_sc as plsc

N_NODES = 10000
N_HE = 5000
N_INC = 320000
D = 128

NC = 2
NS = 16
NW = NC * NS

CHUNK = 128
E_PAD = 327680
E_HALF = 176128
SEG_PAD = 5120
N_SINK = SEG_PAD - 5000


def _ln_v(x, g, b):
    m = jnp.mean(x, axis=-1, keepdims=True)
    v = jnp.mean(jnp.square(x - m), axis=-1, keepdims=True)
    return (x - m) / jnp.sqrt(v + 1e-5) * g + b


def _mlp_v(x, g0, b0, W0, bb0, g1, b1, W1, bb1):
    h = _ln_v(x, g0, b0)
    h = jnp.maximum(jnp.dot(h, W0, preferred_element_type=jnp.float32) + bb0, 0.0)
    h = _ln_v(h, g1, b1)
    return jnp.dot(h, W1, preferred_element_type=jnp.float32) + bb1


def _head_v(x, W0, b0, W1, b1, softplus):
    y = jnp.maximum(jnp.dot(x, W0, preferred_element_type=jnp.float32) + b0, 0.0)
    y = jnp.dot(y, W1, preferred_element_type=jnp.float32) + b1
    return jax.nn.softplus(y) if softplus else y


def _mlp_args(p):
    return (p['ln0_g'].reshape(1, -1), p['ln0_b'].reshape(1, -1),
            p['W0'], p['b0'].reshape(1, -1),
            p['ln1_g'].reshape(1, -1), p['ln1_b'].reshape(1, -1),
            p['W1'], p['b1'].reshape(1, -1))


def _head_args(p):
    return (p['W0'], p['b0'].reshape(1, -1), p['W1'], p['b1'].reshape(1, -1))


def _rep_spec(a):
    return pl.BlockSpec(a.shape, lambda i: (0,) * a.ndim)


def _enc_stage(x, enc_p, blk=1000):
    n = x.shape[0]
    args = _mlp_args(enc_p)

    def body(x_ref, g0, b0, W0, bb0, g1, b1, W1, bb1, o_ref):
        h = _mlp_v(x_ref[...], g0[...], b0[...], W0[...], bb0[...],
                   g1[...], b1[...], W1[...], bb1[...])
        o_ref[...] = jnp.maximum(h, 0.0)

    return pl.pallas_call(
        body, grid=(n // blk,),
        in_specs=[pl.BlockSpec((blk, D), lambda i: (i, 0))] +
                 [_rep_spec(a) for a in args],
        out_specs=pl.BlockSpec((blk, D), lambda i: (i, 0)),
        out_shape=jax.ShapeDtypeStruct((n, D), jnp.float32),
    )(x, *args)


def _he_stage(f0, f1, c0, c1, noise, dec_p, enc_p, mh_p, sh_p, blk=1000):
    n = f0.shape[0]
    args = (_mlp_args(dec_p) + _mlp_args(enc_p) +
            _head_args(mh_p) + _head_args(sh_p))

    def body(f0r, f1r, c0r, c1r, nz, *refs):
        d, e = refs[:8], refs[8:16]
        m, s = refs[16:20], refs[20:24]
        o_enc, o_fin, o_mean, o_std = refs[24:]
        a = f0r[...] + f1r[...]
        c = c0r[...] + c1r[...]
        mean = a / jnp.maximum(c, 1.0)
        x_he = jnp.maximum(_mlp_v(mean, *[r[...] for r in d]), 0.0)
        o_enc[...] = jnp.maximum(_mlp_v(x_he, *[r[...] for r in e]), 0.0)
        mhead = _head_v(x_he, *[r[...] for r in m], softplus=False)
        shead = _head_v(x_he, *[r[...] for r in s], softplus=True)
        o_fin[...] = nz[...] * shead + mhead
        o_mean[...] = mhead
        o_std[...] = shead

    row = pl.BlockSpec((blk, D), lambda i: (i, 0))
    cnt = pl.BlockSpec((blk, 1), lambda i: (i, 0))
    return pl.pallas_call(
        body, grid=(n // blk,),
        in_specs=[row, row, cnt, cnt, row] + [_rep_spec(a) for a in args],
        out_specs=[row, row, row, row],
        out_shape=[jax.ShapeDtypeStruct((n, D), jnp.float32)] * 4,
    )(f0, f1, c0, c1, noise, *args)


def _node_stage(f, c_in, noise, dec_p, mh_p, sh_p, blk=1000):
    n = f.shape[0]
    args = _mlp_args(dec_p) + _head_args(mh_p) + _head_args(sh_p)

    def body(fr, cr, nz, *refs):
        d = refs[:8]
        m, s = refs[8:12], refs[12:16]
        o_fin, o_mean, o_std = refs[16:]
        c = cr[...]
        mean = fr[...] / jnp.maximum(c, 1.0)
        x_node = jnp.maximum(_mlp_v(mean, *[r[...] for r in d]), 0.0)
        mhead = _head_v(x_node, *[r[...] for r in m], softplus=False)
        shead = _head_v(x_node, *[r[...] for r in s], softplus=True)
        o_fin[...] = nz[...] * shead + mhead
        o_mean[...] = mhead
        o_std[...] = shead

    row = pl.BlockSpec((blk, D), lambda i: (i, 0))
    cnt = pl.BlockSpec((blk, 1), lambda i: (i, 0))
    return pl.pallas_call(
        body, grid=(n // blk,),
        in_specs=[row, cnt, row] + [_rep_spec(a) for a in args],
        out_specs=[row, row, row],
        out_shape=[jax.ShapeDtypeStruct((n, D), jnp.float32)] * 3,
    )(f, c_in, noise, *args)


def _make_sc_scatter(mode):
    if mode == "edge":
        e_per_w = E_PAD // NW
    else:
        e_per_w = E_HALF // NS
    n_chunks = e_per_w // CHUNK
    z_rows = SEG_PAD // NS
    mesh = plsc.VectorSubcoreMesh(core_axis_name="c", subcore_axis_name="s")

    @functools.partial(
        pl.kernel,
        mesh=mesh,
        out_type=(
            jax.ShapeDtypeStruct((NC, SEG_PAD, D), jnp.float32),
            jax.ShapeDtypeStruct((NC * SEG_PAD,), jnp.float32),
        ),
        scratch_types=[
            pltpu.VMEM((CHUNK,), jnp.int32),
            pltpu.VMEM((CHUNK,), jnp.int32),
            pltpu.VMEM((CHUNK,), jnp.int32),
            pltpu.VMEM((CHUNK,), jnp.int32),
            pltpu.VMEM((CHUNK, 16), jnp.float32),
            pltpu.VMEM((CHUNK, 16), jnp.float32),
            pltpu.VMEM((CHUNK, D), jnp.float32),
            pltpu.VMEM((CHUNK, D), jnp.float32),
            pltpu.VMEM((CHUNK,), jnp.float32),
            pltpu.VMEM((z_rows,), jnp.float32),
            pltpu.VMEM_SHARED((SEG_PAD, D), jnp.float32),
            pltpu.VMEM_SHARED((SEG_PAD,), jnp.float32),
            pltpu.SemaphoreType.DMA,
            pltpu.SemaphoreType.DMA,
        ],
    )
    def sck(tab, gidx, sidx2, nrm, feats_out, cnt_out,
            gb0, gb1, sb0, sb1, nr0, nr1, rw0, rw1, ones1, cbuf,
            facc, cacc, sm0, sm1):
        cid = lax.axis_index("c")
        sid = lax.axis_index("s")
        if mode == "edge":
            ebase = (sid * NC + cid) * e_per_w
            soff = cid * E_PAD
        else:
            ebase = cid * E_HALF + sid * e_per_w
            soff = 0
        gb = (gb0, gb1)
        sb = (sb0, sb1)
        nr = (nr0, nr1)
        rw = (rw0, rw1)
        sm = (sm0, sm1)

        zero16 = jnp.zeros((16,), jnp.float32)
        one16 = jnp.ones((16,), jnp.float32)

        def zrow(i, _):
            for j in range(D // 16):
                rw0[i, pl.ds(j * 16, 16)] = zero16
            return 0

        lax.fori_loop(0, CHUNK, zrow, 0)

        def zrow1(i, _):
            cbuf[pl.ds(i * 16, 16)] = zero16
            return 0

        lax.fori_loop(0, z_rows // 16, zrow1, 0)

        def orow1(i, _):
            ones1[pl.ds(i * 16, 16)] = one16
            return 0

        lax.fori_loop(0, CHUNK // 16, orow1, 0)

        zb = sid * z_rows
        r = 0
        while r < z_rows:
            nrows = min(CHUNK, z_rows - r)
            pltpu.sync_copy(rw0.at[pl.ds(0, nrows)],
                            facc.at[pl.ds(zb + r, nrows)])
            r += nrows
        pltpu.sync_copy(cbuf, cacc.at[pl.ds(zb, z_rows)])
        plsc.subcore_barrier()

        def stage_and_fire(k, b):
            base = pl.multiple_of(ebase + k * CHUNK, CHUNK)
            sbase = pl.multiple_of(soff + ebase + k * CHUNK, 8)
            pltpu.sync_copy(gidx.at[pl.ds(base, CHUNK)], gb[b])
            pltpu.sync_copy(sidx2.at[pl.ds(sbase, CHUNK)], sb[b])
            pltpu.sync_copy(nrm.at[pl.ds(base, CHUNK)], nr[b])
            pltpu.async_copy(tab.at[gb[b]], rw[b], sm[b])

        def process(b):
            pltpu.make_async_copy(tab.at[gb[b]], rw[b], sm[b]).wait()

            def scale(e, _):
                nsplat = nr[b][e]
                for j in range(D // 16):
                    rw[b][e, pl.ds(j * 16, 16)] = (
                        rw[b][e, pl.ds(j * 16, 16)] * nsplat)
                return 0

            lax.fori_loop(0, CHUNK, scale, 0)
            pltpu.sync_copy(rw[b], facc.at[sb[b]], add=True)
            pltpu.sync_copy(ones1, cacc.at[sb[b]], add=True)

        stage_and_fire(0, 0)

        def pair(kk, _):
            k0 = 2 * kk
            stage_and_fire(k0 + 1, 1)
            process(0)

            @pl.when(kk < n_chunks // 2 - 1)
            def _():
                stage_and_fire(k0 + 2, 0)

            process(1)
            return 0

        lax.fori_loop(0, n_chunks // 2, pair, 0)
        plsc.subcore_barrier()

        pltpu.sync_copy(facc.at[pl.ds(zb, z_rows)],
                        feats_out.at[cid, pl.ds(zb, z_rows)])
        cb = pl.multiple_of(cid * SEG_PAD + zb, 8)
        pltpu.sync_copy(cacc.at[pl.ds(zb, z_rows)], cbuf)
        pltpu.sync_copy(cbuf, cnt_out.at[pl.ds(cb, z_rows)])

    return sck


_sc_scatter_he = _make_sc_scatter("edge")
_sc_scatter_node = _make_sc_scatter("part")


def kernel(x, edge_index, norm, params):
    src = edge_index[0].astype(jnp.int32)
    he = (edge_index[1] - jnp.min(edge_index[1])).astype(jnp.int32)

    npad = E_PAD - N_INC
    zero_pad = jnp.zeros((npad,), jnp.int32)
    valid = jnp.concatenate([jnp.ones((N_INC,), jnp.bool_),
                             jnp.zeros((npad,), jnp.bool_)])
    sink = 5000 + (jnp.arange(E_PAD, dtype=jnp.int32) % N_SINK)

    src_p = jnp.concatenate([src, zero_pad])
    he_p = jnp.concatenate([he, zero_pad])
    he_s = jnp.where(valid, he_p, sink)
    sidx_he = jnp.concatenate([he_s, he_s])
    nrm_p = jnp.concatenate([norm, jnp.zeros((npad,), jnp.float32)])
    nrm_rep = jnp.broadcast_to(nrm_p[:, None], (E_PAD, 16))

    x_enc = _enc_stage(x, params['v2e']['f_enc'])
    f_he, c_he = _sc_scatter_he(x_enc, src_p, sidx_he, nrm_rep)
    c_he = c_he.reshape(NC, SEG_PAD)

    noise_e = jax.random.normal(jax.random.key(8), (N_HE, D), dtype=jnp.float32)
    x_he_enc, he_final, he_mean, he_std = _he_stage(
        f_he[0, :N_HE], f_he[1, :N_HE],
        c_he[0, :N_HE, None], c_he[1, :N_HE, None],
        noise_e, params['v2e']['f_dec'], params['e2v']['f_enc'],
        params['mean_he'], params['std_he'])

    left = src < 5000
    li = left.astype(jnp.int32)
    pos = jnp.where(left, jnp.cumsum(li) - 1,
                    E_HALF + jnp.cumsum(1 - li) - 1)
    zero2 = jnp.zeros((2 * E_HALF,), jnp.int32)
    hp = zero2.at[pos].set(he, unique_indices=True)
    srcp = zero2.at[pos].set(src, unique_indices=True)
    vm = zero2.at[pos].set(1, unique_indices=True)
    nrmp = jnp.zeros((2 * E_HALF,), jnp.float32).at[pos].set(
        norm, unique_indices=True)
    slot = jnp.arange(2 * E_HALF, dtype=jnp.int32)
    sink2 = 5000 + slot % N_SINK
    sidx_nd = jnp.where(vm == 1,
                        jnp.where(slot >= E_HALF, srcp - 5000, srcp), sink2)
    nrm_rep2 = jnp.broadcast_to(nrmp[:, None], (2 * E_HALF, 16))
    f_nd, c_nd = _sc_scatter_node(x_he_enc, hp, sidx_nd, nrm_rep2)
    f_nodes = f_nd[:, :5000].reshape(N_NODES, D)
    c_nodes = c_nd.reshape(NC, SEG_PAD)[:, :5000].reshape(N_NODES, 1)

    noise_n = jax.random.normal(jax.random.key(7), (N_NODES, D),
                                dtype=jnp.float32)
    node_final, node_mean, node_std = _node_stage(
        f_nodes, c_nodes, noise_n, params['e2v']['f_dec'],
        params['mean_node'], params['std_node'])

    return (node_final, node_mean, node_std, he_final, he_mean, he_std)


# speedup vs baseline: 3.5516x; 3.5516x over previous
import functools

import jax
import jax.numpy as jnp
from jax import lax
from jax.experimental import pallas as pl
from jax.experimental.pallas import tpu as pltpu
from jax.experimental.pallas import tpu_sc as plsc

N_NODES = 10000
N_HE = 5000
N_INC = 320000
D = 128

NC = 2
NS = 16
NW = NC * NS

CHUNK = 128
E_PAD = 327680
E_HALF = 176128
SEG_PAD = 5120
N_SINK = SEG_PAD - 5000


def _ln_v(x, g, b):
    m = jnp.mean(x, axis=-1, keepdims=True)
    v = jnp.mean(jnp.square(x - m), axis=-1, keepdims=True)
    return (x - m) / jnp.sqrt(v + 1e-5) * g + b


def _mlp_v(x, g0, b0, W0, bb0, g1, b1, W1, bb1):
    h = _ln_v(x, g0, b0)
    h = jnp.maximum(jnp.dot(h, W0, preferred_element_type=jnp.float32) + bb0, 0.0)
    h = _ln_v(h, g1, b1)
    return jnp.dot(h, W1, preferred_element_type=jnp.float32) + bb1


def _head_v(x, W0, b0, W1, b1, softplus):
    y = jnp.maximum(jnp.dot(x, W0, preferred_element_type=jnp.float32) + b0, 0.0)
    y = jnp.dot(y, W1, preferred_element_type=jnp.float32) + b1
    return jax.nn.softplus(y) if softplus else y


def _mlp_args(p):
    return (p['ln0_g'].reshape(1, -1), p['ln0_b'].reshape(1, -1),
            p['W0'], p['b0'].reshape(1, -1),
            p['ln1_g'].reshape(1, -1), p['ln1_b'].reshape(1, -1),
            p['W1'], p['b1'].reshape(1, -1))


def _head_args(p):
    return (p['W0'], p['b0'].reshape(1, -1), p['W1'], p['b1'].reshape(1, -1))


def _rep_spec(a):
    return pl.BlockSpec(a.shape, lambda i: (0,) * a.ndim)


def _enc_stage(x, enc_p, blk=1000):
    n = x.shape[0]
    args = _mlp_args(enc_p)

    def body(x_ref, g0, b0, W0, bb0, g1, b1, W1, bb1, o_ref):
        h = _mlp_v(x_ref[...], g0[...], b0[...], W0[...], bb0[...],
                   g1[...], b1[...], W1[...], bb1[...])
        o_ref[...] = jnp.maximum(h, 0.0)

    return pl.pallas_call(
        body, grid=(n // blk,),
        in_specs=[pl.BlockSpec((blk, D), lambda i: (i, 0))] +
                 [_rep_spec(a) for a in args],
        out_specs=pl.BlockSpec((blk, D), lambda i: (i, 0)),
        out_shape=jax.ShapeDtypeStruct((n, D), jnp.float32),
    )(x, *args)


def _he_stage(f0, f1, c0, c1, noise, dec_p, enc_p, mh_p, sh_p, blk=1000):
    n = f0.shape[0]
    args = (_mlp_args(dec_p) + _mlp_args(enc_p) +
            _head_args(mh_p) + _head_args(sh_p))

    def body(f0r, f1r, c0r, c1r, nz, *refs):
        d, e = refs[:8], refs[8:16]
        m, s = refs[16:20], refs[20:24]
        o_enc, o_fin, o_mean, o_std = refs[24:]
        a = f0r[...] + f1r[...]
        c = c0r[...] + c1r[...]
        mean = a / jnp.maximum(c, 1.0)
        x_he = jnp.maximum(_mlp_v(mean, *[r[...] for r in d]), 0.0)
        o_enc[...] = jnp.maximum(_mlp_v(x_he, *[r[...] for r in e]), 0.0)
        mhead = _head_v(x_he, *[r[...] for r in m], softplus=False)
        shead = _head_v(x_he, *[r[...] for r in s], softplus=True)
        o_fin[...] = nz[...] * shead + mhead
        o_mean[...] = mhead
        o_std[...] = shead

    row = pl.BlockSpec((blk, D), lambda i: (i, 0))
    cnt = pl.BlockSpec((blk, 1), lambda i: (i, 0))
    return pl.pallas_call(
        body, grid=(n // blk,),
        in_specs=[row, row, cnt, cnt, row] + [_rep_spec(a) for a in args],
        out_specs=[row, row, row, row],
        out_shape=[jax.ShapeDtypeStruct((n, D), jnp.float32)] * 4,
    )(f0, f1, c0, c1, noise, *args)


def _node_stage(f, c_in, noise, dec_p, mh_p, sh_p, blk=1000):
    n = f.shape[0]
    args = _mlp_args(dec_p) + _head_args(mh_p) + _head_args(sh_p)

    def body(fr, cr, nz, *refs):
        d = refs[:8]
        m, s = refs[8:12], refs[12:16]
        o_fin, o_mean, o_std = refs[16:]
        c = cr[...]
        mean = fr[...] / jnp.maximum(c, 1.0)
        x_node = jnp.maximum(_mlp_v(mean, *[r[...] for r in d]), 0.0)
        mhead = _head_v(x_node, *[r[...] for r in m], softplus=False)
        shead = _head_v(x_node, *[r[...] for r in s], softplus=True)
        o_fin[...] = nz[...] * shead + mhead
        o_mean[...] = mhead
        o_std[...] = shead

    row = pl.BlockSpec((blk, D), lambda i: (i, 0))
    cnt = pl.BlockSpec((blk, 1), lambda i: (i, 0))
    return pl.pallas_call(
        body, grid=(n // blk,),
        in_specs=[row, cnt, row] + [_rep_spec(a) for a in args],
        out_specs=[row, row, row],
        out_shape=[jax.ShapeDtypeStruct((n, D), jnp.float32)] * 3,
    )(f, c_in, noise, *args)


def _make_sc_scatter(mode):
    e_per_w = E_PAD // (NW if mode == "edge" else NS)
    n_chunks = e_per_w // CHUNK
    z_rows = SEG_PAD // NS
    mesh = plsc.VectorSubcoreMesh(core_axis_name="c", subcore_axis_name="s")

    @functools.partial(
        pl.kernel,
        mesh=mesh,
        out_type=(
            jax.ShapeDtypeStruct((NC, SEG_PAD, D), jnp.float32),
            jax.ShapeDtypeStruct((NC * SEG_PAD,), jnp.float32),
        ),
        scratch_types=[
            pltpu.VMEM((CHUNK,), jnp.int32),
            pltpu.VMEM((CHUNK,), jnp.int32),
            pltpu.VMEM((CHUNK,), jnp.int32),
            pltpu.VMEM((CHUNK,), jnp.int32),
            pltpu.VMEM((CHUNK, 16), jnp.float32),
            pltpu.VMEM((CHUNK, 16), jnp.float32),
            pltpu.VMEM((CHUNK, D), jnp.float32),
            pltpu.VMEM((CHUNK, D), jnp.float32),
            pltpu.VMEM((CHUNK,), jnp.float32),
            pltpu.VMEM((z_rows,), jnp.float32),
            pltpu.VMEM_SHARED((SEG_PAD, D), jnp.float32),
            pltpu.VMEM_SHARED((SEG_PAD,), jnp.float32),
            pltpu.SemaphoreType.DMA,
            pltpu.SemaphoreType.DMA,
            pltpu.SemaphoreType.DMA,
            pltpu.SemaphoreType.DMA,
        ],
    )
    def sck(tab, gidx, sidx2, nrm, feats_out, cnt_out,
            gb0, gb1, sb0, sb1, nr0, nr1, rw0, rw1, ones1, cbuf,
            facc, cacc, sm0, sm1, sm2, sm3):
        cid = lax.axis_index("c")
        sid = lax.axis_index("s")
        if mode == "edge":
            ebase = (sid * NC + cid) * e_per_w
        else:
            ebase = sid * e_per_w
        soff = cid * E_PAD
        gb = (gb0, gb1)
        sb = (sb0, sb1)
        nr = (nr0, nr1)
        rw = (rw0, rw1)
        sm = (sm0, sm1)
        sm_hi = (sm2, sm3)

        zero16 = jnp.zeros((16,), jnp.float32)
        one16 = jnp.ones((16,), jnp.float32)

        def zrow(i, _):
            for j in range(D // 16):
                rw0[i, pl.ds(j * 16, 16)] = zero16
            return 0

        lax.fori_loop(0, CHUNK, zrow, 0)

        def zrow1(i, _):
            cbuf[pl.ds(i * 16, 16)] = zero16
            return 0

        lax.fori_loop(0, z_rows // 16, zrow1, 0)

        def orow1(i, _):
            ones1[pl.ds(i * 16, 16)] = one16
            return 0

        lax.fori_loop(0, CHUNK // 16, orow1, 0)

        zb = sid * z_rows
        r = 0
        while r < z_rows:
            nrows = min(CHUNK, z_rows - r)
            pltpu.sync_copy(rw0.at[pl.ds(0, nrows)],
                            facc.at[pl.ds(zb + r, nrows)])
            r += nrows
        pltpu.sync_copy(cbuf, cacc.at[pl.ds(zb, z_rows)])
        plsc.subcore_barrier()

        def stage_and_fire(k, b):
            base = pl.multiple_of(ebase + k * CHUNK, CHUNK)
            sbase = pl.multiple_of(soff + ebase + k * CHUNK, 8)
            pltpu.sync_copy(gidx.at[pl.ds(base, CHUNK)], gb[b])
            pltpu.sync_copy(sidx2.at[pl.ds(sbase, CHUNK)], sb[b])
            pltpu.sync_copy(nrm.at[pl.ds(base, CHUNK)], nr[b])
            pltpu.async_copy(tab.at[gb[b].at[pl.ds(0, CHUNK // 2)]],
                             rw[b].at[pl.ds(0, CHUNK // 2)], sm[b])
            pltpu.async_copy(tab.at[gb[b].at[pl.ds(CHUNK // 2, CHUNK // 2)]],
                             rw[b].at[pl.ds(CHUNK // 2, CHUNK // 2)], sm_hi[b])

        def process(b):
            pltpu.make_async_copy(tab.at[gb[b].at[pl.ds(0, CHUNK // 2)]],
                                  rw[b].at[pl.ds(0, CHUNK // 2)], sm[b]).wait()
            pltpu.make_async_copy(tab.at[gb[b].at[pl.ds(CHUNK // 2, CHUNK // 2)]],
                                  rw[b].at[pl.ds(CHUNK // 2, CHUNK // 2)],
                                  sm_hi[b]).wait()

            def scale(e, _):
                nsplat = nr[b][e]
                for j in range(D // 16):
                    rw[b][e, pl.ds(j * 16, 16)] = (
                        rw[b][e, pl.ds(j * 16, 16)] * nsplat)
                return 0

            lax.fori_loop(0, CHUNK, scale, 0)
            pltpu.sync_copy(rw[b], facc.at[sb[b]], add=True)
            pltpu.sync_copy(ones1, cacc.at[sb[b]], add=True)

        stage_and_fire(0, 0)

        def pair(kk, _):
            k0 = 2 * kk
            stage_and_fire(k0 + 1, 1)
            process(0)

            @pl.when(kk < n_chunks // 2 - 1)
            def _():
                stage_and_fire(k0 + 2, 0)

            process(1)
            return 0

        lax.fori_loop(0, n_chunks // 2, pair, 0)
        plsc.subcore_barrier()

        pltpu.sync_copy(facc.at[pl.ds(zb, z_rows)],
                        feats_out.at[cid, pl.ds(zb, z_rows)])
        cb = pl.multiple_of(cid * SEG_PAD + zb, 8)
        pltpu.sync_copy(cacc.at[pl.ds(zb, z_rows)], cbuf)
        pltpu.sync_copy(cbuf, cnt_out.at[pl.ds(cb, z_rows)])

    return sck


_sc_scatter_he = _make_sc_scatter("edge")
_sc_scatter_node = _make_sc_scatter("seg")


def kernel(x, edge_index, norm, params):
    src = edge_index[0].astype(jnp.int32)
    he = (edge_index[1] - jnp.min(edge_index[1])).astype(jnp.int32)

    npad = E_PAD - N_INC
    zero_pad = jnp.zeros((npad,), jnp.int32)
    valid = jnp.concatenate([jnp.ones((N_INC,), jnp.bool_),
                             jnp.zeros((npad,), jnp.bool_)])
    sink = 5000 + (jnp.arange(E_PAD, dtype=jnp.int32) % N_SINK)

    src_p = jnp.concatenate([src, zero_pad])
    he_p = jnp.concatenate([he, zero_pad])
    he_s = jnp.where(valid, he_p, sink)
    sidx_he = jnp.concatenate([he_s, he_s])
    nrm_p = jnp.concatenate([norm, jnp.zeros((npad,), jnp.float32)])
    nrm_rep = jnp.broadcast_to(nrm_p[:, None], (E_PAD, 16))

    x_enc = _enc_stage(x, params['v2e']['f_enc'])
    f_he, c_he = _sc_scatter_he(x_enc, src_p, sidx_he, nrm_rep)
    c_he = c_he.reshape(NC, SEG_PAD)

    noise_e = jax.random.normal(jax.random.key(8), (N_HE, D), dtype=jnp.float32)
    x_he_enc, he_final, he_mean, he_std = _he_stage(
        f_he[0, :N_HE], f_he[1, :N_HE],
        c_he[0, :N_HE, None], c_he[1, :N_HE, None],
        noise_e, params['v2e']['f_dec'], params['e2v']['f_enc'],
        params['mean_he'], params['std_he'])

    sidx_nd = jnp.concatenate([
        jnp.where(valid & (src_p < 5000), src_p, sink),
        jnp.where(valid & (src_p >= 5000), src_p - 5000, sink),
    ])
    f_nd, c_nd = _sc_scatter_node(x_he_enc, he_p, sidx_nd, nrm_rep)
    f_nodes = f_nd[:, :5000].reshape(N_NODES, D)
    c_nodes = c_nd.reshape(NC, SEG_PAD)[:, :5000].reshape(N_NODES, 1)

    noise_n = jax.random.normal(jax.random.key(7), (N_NODES, D),
                                dtype=jnp.float32)
    node_final, node_mean, node_std = _node_stage(
        f_nodes, c_nodes, noise_n, params['e2v']['f_dec'],
        params['mean_node'], params['std_node'])

    return (node_final, node_mean, node_std, he_final, he_mean, he_std)


# speedup vs baseline: 3.9732x; 1.1187x over previous
import functools

import jax
import jax.numpy as jnp
from jax import lax
from jax.experimental import pallas as pl
from jax.experimental.pallas import tpu as pltpu
from jax.experimental.pallas import tpu_sc as plsc

N_NODES = 10000
N_HE = 5000
N_INC = 320000
D = 128

NC = 2
NS = 16
NW = NC * NS

CHUNK = 128
E_PAD = 327680
E_HALF = 176128
SEG_PAD = 5120
N_SINK = SEG_PAD - 5000


def _ln_v(x, g, b):
    m = jnp.mean(x, axis=-1, keepdims=True)
    v = jnp.mean(jnp.square(x - m), axis=-1, keepdims=True)
    return (x - m) / jnp.sqrt(v + 1e-5) * g + b


def _mlp_v(x, g0, b0, W0, bb0, g1, b1, W1, bb1):
    h = _ln_v(x, g0, b0)
    h = jnp.maximum(jnp.dot(h, W0, preferred_element_type=jnp.float32) + bb0, 0.0)
    h = _ln_v(h, g1, b1)
    return jnp.dot(h, W1, preferred_element_type=jnp.float32) + bb1


def _head_v(x, W0, b0, W1, b1, softplus):
    y = jnp.maximum(jnp.dot(x, W0, preferred_element_type=jnp.float32) + b0, 0.0)
    y = jnp.dot(y, W1, preferred_element_type=jnp.float32) + b1
    return jax.nn.softplus(y) if softplus else y


def _mlp_args(p):
    return (p['ln0_g'].reshape(1, -1), p['ln0_b'].reshape(1, -1),
            p['W0'], p['b0'].reshape(1, -1),
            p['ln1_g'].reshape(1, -1), p['ln1_b'].reshape(1, -1),
            p['W1'], p['b1'].reshape(1, -1))


def _head_args(p):
    return (p['W0'], p['b0'].reshape(1, -1), p['W1'], p['b1'].reshape(1, -1))


def _rep_spec(a):
    return pl.BlockSpec(a.shape, lambda i: (0,) * a.ndim)


def _enc_stage(x, enc_p, blk=1000):
    n = x.shape[0]
    args = _mlp_args(enc_p)

    def body(x_ref, g0, b0, W0, bb0, g1, b1, W1, bb1, o_ref):
        h = _mlp_v(x_ref[...], g0[...], b0[...], W0[...], bb0[...],
                   g1[...], b1[...], W1[...], bb1[...])
        o_ref[...] = jnp.maximum(h, 0.0)

    return pl.pallas_call(
        body, grid=(n // blk,),
        in_specs=[pl.BlockSpec((blk, D), lambda i: (i, 0))] +
                 [_rep_spec(a) for a in args],
        out_specs=pl.BlockSpec((blk, D), lambda i: (i, 0)),
        out_shape=jax.ShapeDtypeStruct((n, D), jnp.float32),
    )(x, *args)


def _he_stage(f0, f1, c0, c1, noise, dec_p, enc_p, mh_p, sh_p, blk=1000):
    n = f0.shape[0]
    args = (_mlp_args(dec_p) + _mlp_args(enc_p) +
            _head_args(mh_p) + _head_args(sh_p))

    def body(f0r, f1r, c0r, c1r, nz, *refs):
        d, e = refs[:8], refs[8:16]
        m, s = refs[16:20], refs[20:24]
        o_enc, o_fin, o_mean, o_std = refs[24:]
        a = f0r[...] + f1r[...]
        c = c0r[...] + c1r[...]
        mean = a / jnp.maximum(c, 1.0)
        x_he = jnp.maximum(_mlp_v(mean, *[r[...] for r in d]), 0.0)
        o_enc[...] = jnp.maximum(_mlp_v(x_he, *[r[...] for r in e]), 0.0)
        mhead = _head_v(x_he, *[r[...] for r in m], softplus=False)
        shead = _head_v(x_he, *[r[...] for r in s], softplus=True)
        o_fin[...] = nz[...] * shead + mhead
        o_mean[...] = mhead
        o_std[...] = shead

    row = pl.BlockSpec((blk, D), lambda i: (i, 0))
    cnt = pl.BlockSpec((blk, 1), lambda i: (i, 0))
    return pl.pallas_call(
        body, grid=(n // blk,),
        in_specs=[row, row, cnt, cnt, row] + [_rep_spec(a) for a in args],
        out_specs=[row, row, row, row],
        out_shape=[jax.ShapeDtypeStruct((n, D), jnp.float32)] * 4,
    )(f0, f1, c0, c1, noise, *args)


def _node_stage(f, c_in, noise, dec_p, mh_p, sh_p, blk=1000):
    n = f.shape[0]
    args = _mlp_args(dec_p) + _head_args(mh_p) + _head_args(sh_p)

    def body(fr, cr, nz, *refs):
        d = refs[:8]
        m, s = refs[8:12], refs[12:16]
        o_fin, o_mean, o_std = refs[16:]
        c = cr[...]
        mean = fr[...] / jnp.maximum(c, 1.0)
        x_node = jnp.maximum(_mlp_v(mean, *[r[...] for r in d]), 0.0)
        mhead = _head_v(x_node, *[r[...] for r in m], softplus=False)
        shead = _head_v(x_node, *[r[...] for r in s], softplus=True)
        o_fin[...] = nz[...] * shead + mhead
        o_mean[...] = mhead
        o_std[...] = shead

    row = pl.BlockSpec((blk, D), lambda i: (i, 0))
    cnt = pl.BlockSpec((blk, 1), lambda i: (i, 0))
    return pl.pallas_call(
        body, grid=(n // blk,),
        in_specs=[row, cnt, row] + [_rep_spec(a) for a in args],
        out_specs=[row, row, row],
        out_shape=[jax.ShapeDtypeStruct((n, D), jnp.float32)] * 3,
    )(f, c_in, noise, *args)


def _make_sc_scatter(mode):
    e_per_w = E_PAD // (NW if mode == "edge" else NS)
    n_chunks = e_per_w // CHUNK
    z_rows = SEG_PAD // NS
    mesh = plsc.VectorSubcoreMesh(core_axis_name="c", subcore_axis_name="s")

    @functools.partial(
        pl.kernel,
        mesh=mesh,
        out_type=(
            jax.ShapeDtypeStruct((NC, SEG_PAD, D), jnp.float32),
            jax.ShapeDtypeStruct((NC * SEG_PAD,), jnp.float32),
        ),
        scratch_types=[
            pltpu.VMEM((CHUNK,), jnp.int32),
            pltpu.VMEM((CHUNK,), jnp.int32),
            pltpu.VMEM((CHUNK,), jnp.int32),
            pltpu.VMEM((CHUNK,), jnp.int32),
            pltpu.VMEM((CHUNK, 16), jnp.float32),
            pltpu.VMEM((CHUNK, 16), jnp.float32),
            pltpu.VMEM((CHUNK, D), jnp.float32),
            pltpu.VMEM((CHUNK, D), jnp.float32),
            pltpu.VMEM((CHUNK,), jnp.float32),
            pltpu.VMEM((z_rows,), jnp.float32),
            pltpu.VMEM_SHARED((SEG_PAD, D), jnp.float32),
            pltpu.VMEM_SHARED((SEG_PAD,), jnp.float32),
            pltpu.SemaphoreType.DMA,
            pltpu.SemaphoreType.DMA,
            pltpu.SemaphoreType.DMA,
            pltpu.SemaphoreType.DMA,
            pltpu.SemaphoreType.DMA,
            pltpu.SemaphoreType.DMA,
        ],
    )
    def sck(tab, gidx, sidx2, nrm, feats_out, cnt_out,
            gb0, gb1, sb0, sb1, nr0, nr1, rw0, rw1, ones1, cbuf,
            facc, cacc, sm0, sm1, st0, st1, sx0, sx1):
        cid = lax.axis_index("c")
        sid = lax.axis_index("s")
        if mode == "edge":
            ebase = (sid * NC + cid) * e_per_w
        else:
            ebase = sid * e_per_w
        soff = cid * E_PAD
        gb = (gb0, gb1)
        sb = (sb0, sb1)
        nr = (nr0, nr1)
        rw = (rw0, rw1)
        sm = (sm0, sm1)
        st = (st0, st1)
        sx = (sx0, sx1)

        zero16 = jnp.zeros((16,), jnp.float32)
        one16 = jnp.ones((16,), jnp.float32)

        def zrow(i, _):
            for j in range(D // 16):
                rw0[i, pl.ds(j * 16, 16)] = zero16
            return 0

        lax.fori_loop(0, CHUNK, zrow, 0)

        def zrow1(i, _):
            cbuf[pl.ds(i * 16, 16)] = zero16
            return 0

        lax.fori_loop(0, z_rows // 16, zrow1, 0)

        def orow1(i, _):
            ones1[pl.ds(i * 16, 16)] = one16
            return 0

        lax.fori_loop(0, CHUNK // 16, orow1, 0)

        zb = sid * z_rows
        r = 0
        while r < z_rows:
            nrows = min(CHUNK, z_rows - r)
            pltpu.sync_copy(rw0.at[pl.ds(0, nrows)],
                            facc.at[pl.ds(zb + r, nrows)])
            r += nrows
        pltpu.sync_copy(cbuf, cacc.at[pl.ds(zb, z_rows)])
        plsc.subcore_barrier()

        def prestage(k, b):
            base = pl.multiple_of(ebase + k * CHUNK, CHUNK)
            sbase = pl.multiple_of(soff + ebase + k * CHUNK, 8)
            pltpu.async_copy(gidx.at[pl.ds(base, CHUNK)], gb[b], st[b])
            pltpu.async_copy(sidx2.at[pl.ds(sbase, CHUNK)], sb[b], sx[b])
            pltpu.async_copy(nrm.at[pl.ds(base, CHUNK)], nr[b], sx[b])

        def firegather(k, b):
            base = pl.multiple_of(ebase + k * CHUNK, CHUNK)
            pltpu.make_async_copy(gidx.at[pl.ds(base, CHUNK)], gb[b],
                                  st[b]).wait()
            pltpu.async_copy(tab.at[gb[b]], rw[b], sm[b])

        def process(k, b):
            base = pl.multiple_of(ebase + k * CHUNK, CHUNK)
            sbase = pl.multiple_of(soff + ebase + k * CHUNK, 8)
            pltpu.make_async_copy(tab.at[gb[b]], rw[b], sm[b]).wait()
            pltpu.make_async_copy(sidx2.at[pl.ds(sbase, CHUNK)], sb[b],
                                  sx[b]).wait()
            pltpu.make_async_copy(nrm.at[pl.ds(base, CHUNK)], nr[b],
                                  sx[b]).wait()

            def scale(e, _):
                nsplat = nr[b][e]
                for j in range(D // 16):
                    rw[b][e, pl.ds(j * 16, 16)] = (
                        rw[b][e, pl.ds(j * 16, 16)] * nsplat)
                return 0

            lax.fori_loop(0, CHUNK, scale, 0)
            pltpu.sync_copy(rw[b], facc.at[sb[b]], add=True)
            pltpu.sync_copy(ones1, cacc.at[sb[b]], add=True)

        prestage(0, 0)
        firegather(0, 0)
        prestage(1, 1)

        def pair(kk, _):
            k0 = 2 * kk
            firegather(k0 + 1, 1)
            process(k0, 0)

            @pl.when(kk < n_chunks // 2 - 1)
            def _():
                prestage(k0 + 2, 0)
                firegather(k0 + 2, 0)

            process(k0 + 1, 1)

            @pl.when(kk < n_chunks // 2 - 1)
            def _():
                prestage(k0 + 3, 1)
            return 0

        lax.fori_loop(0, n_chunks // 2, pair, 0)
        plsc.subcore_barrier()

        pltpu.sync_copy(facc.at[pl.ds(zb, z_rows)],
                        feats_out.at[cid, pl.ds(zb, z_rows)])
        cb = pl.multiple_of(cid * SEG_PAD + zb, 8)
        pltpu.sync_copy(cacc.at[pl.ds(zb, z_rows)], cbuf)
        pltpu.sync_copy(cbuf, cnt_out.at[pl.ds(cb, z_rows)])

    return sck


_sc_scatter_he = _make_sc_scatter("edge")
_sc_scatter_node = _make_sc_scatter("seg")


def kernel(x, edge_index, norm, params):
    src = edge_index[0].astype(jnp.int32)
    he = (edge_index[1] - jnp.min(edge_index[1])).astype(jnp.int32)

    npad = E_PAD - N_INC
    zero_pad = jnp.zeros((npad,), jnp.int32)
    valid = jnp.concatenate([jnp.ones((N_INC,), jnp.bool_),
                             jnp.zeros((npad,), jnp.bool_)])
    sink = 5000 + (jnp.arange(E_PAD, dtype=jnp.int32) % N_SINK)

    src_p = jnp.concatenate([src, zero_pad])
    he_p = jnp.concatenate([he, zero_pad])
    he_s = jnp.where(valid, he_p, sink)
    sidx_he = jnp.concatenate([he_s, he_s])
    nrm_p = jnp.concatenate([norm, jnp.zeros((npad,), jnp.float32)])
    nrm_rep = jnp.broadcast_to(nrm_p[:, None], (E_PAD, 16))

    x_enc = _enc_stage(x, params['v2e']['f_enc'])
    f_he, c_he = _sc_scatter_he(x_enc, src_p, sidx_he, nrm_rep)
    c_he = c_he.reshape(NC, SEG_PAD)

    noise_e = jax.random.normal(jax.random.key(8), (N_HE, D), dtype=jnp.float32)
    x_he_enc, he_final, he_mean, he_std = _he_stage(
        f_he[0, :N_HE], f_he[1, :N_HE],
        c_he[0, :N_HE, None], c_he[1, :N_HE, None],
        noise_e, params['v2e']['f_dec'], params['e2v']['f_enc'],
        params['mean_he'], params['std_he'])

    sidx_nd = jnp.concatenate([
        jnp.where(valid & (src_p < 5000), src_p, sink),
        jnp.where(valid & (src_p >= 5000), src_p - 5000, sink),
    ])
    f_nd, c_nd = _sc_scatter_node(x_he_enc, he_p, sidx_nd, nrm_rep)
    f_nodes = f_nd[:, :5000].reshape(N_NODES, D)
    c_nodes = c_nd.reshape(NC, SEG_PAD)[:, :5000].reshape(N_NODES, 1)

    noise_n = jax.random.normal(jax.random.key(7), (N_NODES, D),
                                dtype=jnp.float32)
    node_final, node_mean, node_std = _node_stage(
        f_nodes, c_nodes, noise_n, params['e2v']['f_dec'],
        params['mean_node'], params['std_node'])

    return (node_final, node_mean, node_std, he_final, he_mean, he_std)
